# baseline (speedup 1.0000x reference)
import functools

import jax
import jax.numpy as jnp
from jax import lax
from jax.experimental import pallas as pl
from jax.experimental.pallas import tpu as pltpu
from jax.experimental.pallas import tpu_sc as plsc

BB = 128
SC_NC = 2
SC_NS = 16
SC_CHUNKS = 48


def _pool_body(ptr_ref, op_ref, hid_ref, vals_ref, nptr_ref):
    vals_ref[...] = jnp.max(hid_ref[...], axis=1)
    a0 = op_ref[:, 0:1]
    a1 = op_ref[:, 1:2]
    a2 = op_ref[:, 2:3]
    am = jnp.where(a1 > a0, 1, 0)
    am = jnp.where(a2 > jnp.maximum(a0, a1), 2, am)
    nptr_ref[...] = jnp.maximum(ptr_ref[...] + am - 1, 0)


def _sc_copy_body(stack_hbm, out_hbm, buf0, buf1, si0, si1, so0, so1):
    wid = lax.axis_index("s") * SC_NC + lax.axis_index("c")
    nw = SC_NC * SC_NS
    per_w = stack_hbm.shape[0] // nw
    cr = per_w // SC_CHUNKS
    r0 = wid * per_w
    bufs = (buf0, buf1)
    isems = (si0, si1)
    osems = (so0, so1)

    def in_copy(c, k):
        return pltpu.make_async_copy(
            stack_hbm.at[pl.ds(r0 + c * cr, cr)], bufs[k], isems[k])

    def out_copy(c, k):
        return pltpu.make_async_copy(
            bufs[k], out_hbm.at[pl.ds(r0 + c * cr, cr)], osems[k])

    for c in range(SC_CHUNKS):
        k = c % 2
        if c >= 2:
            out_copy(c - 2, k).wait()
        in_copy(c, k).start()
        in_copy(c, k).wait()
        out_copy(c, k).start()
    out_copy(SC_CHUNKS - 2, 0).wait()
    out_copy(SC_CHUNKS - 1, 1).wait()


def _patch_body(stack_in_ref, vals_ref, ptr_ref, out_ref, sem):
    del stack_in_ref
    n = vals_ref.shape[0]

    def start_one(i, carry):
        p = ptr_ref[i]
        pltpu.make_async_copy(
            vals_ref.at[pl.ds(i, 1)],
            out_ref.at[pl.ds((p + 1) * n + i, 1)],
            sem,
        ).start()
        return carry

    lax.fori_loop(0, n, start_one, 0)

    def wait_one(i, carry):
        pltpu.make_async_copy(
            vals_ref.at[pl.ds(0, 1)],
            out_ref.at[pl.ds(0, 1)],
            sem,
        ).wait()
        return carry

    lax.fori_loop(0, n, wait_one, 0)


def kernel(stack, stack_pointers, stack_op, hiddens, graph_fts):
    del graph_fts
    B, T1, Hs = stack.shape
    NN = hiddens.shape[1]
    ptr2 = stack_pointers.reshape(B, 1)
    stack_flat = jnp.transpose(stack, (1, 0, 2)).reshape(T1 * B, Hs)

    vals, nptr = pl.pallas_call(
        _pool_body,
        grid=(B // BB,),
        in_specs=[
            pl.BlockSpec((BB, 1), lambda i: (i, 0)),
            pl.BlockSpec((BB, 3), lambda i: (i, 0)),
            pl.BlockSpec((BB, NN, Hs), lambda i: (i, 0, 0)),
        ],
        out_specs=[
            pl.BlockSpec((BB, Hs), lambda i: (i, 0)),
            pl.BlockSpec((BB, 1), lambda i: (i, 0)),
        ],
        out_shape=[
            jax.ShapeDtypeStruct((B, Hs), jnp.float32),
            jax.ShapeDtypeStruct((B, 1), jnp.int32),
        ],
    )(ptr2, stack_op, hiddens)

    cr = T1 * B // (SC_NC * SC_NS) // SC_CHUNKS
    sc_copy = functools.partial(
        pl.kernel,
        out_type=jax.ShapeDtypeStruct((T1 * B, Hs), stack.dtype),
        mesh=plsc.VectorSubcoreMesh(core_axis_name="c", subcore_axis_name="s"),
        scratch_types=[
            pltpu.VMEM((cr, Hs), jnp.float32),
            pltpu.VMEM((cr, Hs), jnp.float32),
            pltpu.SemaphoreType.DMA,
            pltpu.SemaphoreType.DMA,
            pltpu.SemaphoreType.DMA,
            pltpu.SemaphoreType.DMA,
        ],
    )(_sc_copy_body)
    out_c = sc_copy(stack_flat)

    out_flat = pl.pallas_call(
        _patch_body,
        in_specs=[
            pl.BlockSpec(memory_space=pl.ANY),
            pl.BlockSpec(memory_space=pltpu.VMEM),
            pl.BlockSpec(memory_space=pltpu.SMEM),
        ],
        out_specs=pl.BlockSpec(memory_space=pl.ANY),
        out_shape=jax.ShapeDtypeStruct((T1 * B, Hs), stack.dtype),
        scratch_shapes=[pltpu.SemaphoreType.DMA],
        input_output_aliases={0: 0},
    )(out_c, vals, stack_pointers)

    return jnp.transpose(out_flat.reshape(T1, B, Hs), (1, 0, 2)), nptr.reshape(B)


# speedup vs baseline: 1.3891x; 1.3891x over previous
import jax
import jax.numpy as jnp
from jax import lax
from jax.experimental import pallas as pl

BB = 128


def _body(stack_ref, ptr_ref, op_ref, hid_ref, out_ref, nptr_ref):
    T1 = stack_ref.shape[0]
    vals = jnp.max(hid_ref[...], axis=1)
    ptr = ptr_ref[...]
    stepi = lax.broadcasted_iota(jnp.int32, (T1, BB, 1), 0)
    sel = stepi == (ptr + 1).reshape(1, BB, 1)
    out_ref[...] = jnp.where(sel, vals[None, :, :], stack_ref[...])
    a0 = op_ref[:, 0:1]
    a1 = op_ref[:, 1:2]
    a2 = op_ref[:, 2:3]
    am = jnp.where(a1 > a0, 1, 0)
    am = jnp.where(a2 > jnp.maximum(a0, a1), 2, am)
    nptr_ref[...] = jnp.maximum(ptr + am - 1, 0)


def kernel(stack, stack_pointers, stack_op, hiddens, graph_fts):
    del graph_fts
    B, T1, Hs = stack.shape
    NN = hiddens.shape[1]
    stack_t = jnp.transpose(stack, (1, 0, 2))
    ptr2 = stack_pointers.reshape(B, 1)

    out_t, nptr = pl.pallas_call(
        _body,
        grid=(B // BB,),
        in_specs=[
            pl.BlockSpec((T1, BB, Hs), lambda i: (0, i, 0)),
            pl.BlockSpec((BB, 1), lambda i: (i, 0)),
            pl.BlockSpec((BB, 3), lambda i: (i, 0)),
            pl.BlockSpec((BB, NN, Hs), lambda i: (i, 0, 0)),
        ],
        out_specs=[
            pl.BlockSpec((T1, BB, Hs), lambda i: (0, i, 0)),
            pl.BlockSpec((BB, 1), lambda i: (i, 0)),
        ],
        out_shape=[
            jax.ShapeDtypeStruct((T1, B, Hs), stack.dtype),
            jax.ShapeDtypeStruct((B, 1), jnp.int32),
        ],
    )(stack_t, ptr2, stack_op, hiddens)

    return jnp.transpose(out_t, (1, 0, 2)), nptr.reshape(B)
